# baseline (speedup 1.0000x reference)
import functools

import jax
import jax.numpy as jnp
from jax import lax
from jax.experimental import pallas as pl
from jax.experimental.pallas import tpu as pltpu
from jax.experimental.pallas import tpu_sc as plsc

D_MODEL = 64
SCALE = 8.0

_info = plsc.get_sparse_core_info()
_NC, _NS = _info.num_cores, _info.num_subcores
_NW = _NC * _NS

IDX_PER_ROW = 128
R = 4


def _make_gather(n_rows):
    rows_per_w = n_rows // _NW
    chunks = rows_per_w // R
    mesh = plsc.VectorSubcoreMesh(core_axis_name="c", subcore_axis_name="s")

    @functools.partial(
        pl.kernel,
        mesh=mesh,
        compiler_params=pltpu.CompilerParams(use_tc_tiling_on_sc=False),
        out_type=jax.ShapeDtypeStruct((n_rows * IDX_PER_ROW, D_MODEL), jnp.float32),
        scratch_types=[
            pltpu.VMEM((R, IDX_PER_ROW), jnp.int32),
            pltpu.VMEM((R * IDX_PER_ROW, D_MODEL), jnp.float32),
            pltpu.SemaphoreType.DMA,
        ],
    )
    def gather_scale(idx_hbm, table_hbm, out_hbm, idx_v, rows_v, sem):
        wid = lax.axis_index("s") * _NC + lax.axis_index("c")
        row0 = wid * rows_per_w

        def chunk_body(g, carry):
            r0 = row0 + g * R
            pltpu.sync_copy(idx_hbm.at[pl.ds(r0, R)], idx_v)
            copies = [
                pltpu.async_copy(
                    table_hbm.at[idx_v.at[r]],
                    rows_v.at[pl.ds(r * IDX_PER_ROW, IDX_PER_ROW)],
                    sem,
                )
                for r in range(R)
            ]
            for c in copies:
                c.wait()

            def mul_body(i, carry2):
                for j in range(D_MODEL // 16):
                    sl = pl.ds(j * 16, 16)
                    rows_v[i, sl] = rows_v[i, sl] * SCALE
                return carry2

            lax.fori_loop(0, R * IDX_PER_ROW, mul_body, 0, unroll=4)
            pltpu.sync_copy(
                rows_v, out_hbm.at[pl.ds(r0 * IDX_PER_ROW, R * IDX_PER_ROW)]
            )
            return carry

        lax.fori_loop(0, chunks, chunk_body, 0)

    return gather_scale


def kernel(x, lut):
    b, t = x.shape
    flat = x.reshape(-1).astype(jnp.int32)
    n_rows = flat.shape[0] // IDX_PER_ROW
    idx = flat.reshape(n_rows, IDX_PER_ROW)
    out = _make_gather(n_rows)(idx, lut)
    return out.reshape(b, t, D_MODEL)


# speedup vs baseline: 1.0940x; 1.0940x over previous
import functools

import jax
import jax.numpy as jnp
from jax import lax
from jax.experimental import pallas as pl
from jax.experimental.pallas import tpu as pltpu
from jax.experimental.pallas import tpu_sc as plsc

D_MODEL = 64
SCALE = 8.0

_info = plsc.get_sparse_core_info()
_NC, _NS = _info.num_cores, _info.num_subcores
_NW = _NC * _NS

IDX_PER_ROW = 128
R = 4
CHUNK = R * IDX_PER_ROW


def _make_gather(n_rows):
    rows_per_w = n_rows // _NW
    chunks = rows_per_w // R
    chunk_bytes = CHUNK * D_MODEL * 4
    mesh = plsc.VectorSubcoreMesh(core_axis_name="c", subcore_axis_name="s")

    @functools.partial(
        pl.kernel,
        mesh=mesh,
        compiler_params=pltpu.CompilerParams(use_tc_tiling_on_sc=False),
        out_type=jax.ShapeDtypeStruct((n_rows * IDX_PER_ROW, D_MODEL), jnp.float32),
        scratch_types=[
            pltpu.VMEM((rows_per_w, IDX_PER_ROW), jnp.int32),
            pltpu.VMEM((2, CHUNK, D_MODEL), jnp.float32),
            pltpu.SemaphoreType.DMA,
            pltpu.SemaphoreType.DMA,
            pltpu.SemaphoreType.DMA,
            pltpu.SemaphoreType.DMA,
        ],
    )
    def gather_scale(idx_hbm, table_hbm, out_hbm, idx_v, rows_v, g0, g1, w0, w1):
        wid = lax.axis_index("s") * _NC + lax.axis_index("c")
        row0 = wid * rows_per_w
        sem_g = [g0, g1]
        sem_w = [w0, w1]

        pltpu.sync_copy(idx_hbm.at[pl.ds(row0, rows_per_w)], idx_v)

        def fire_gathers(g, b):
            for r in range(R):
                pltpu.async_copy(
                    table_hbm.at[idx_v.at[g * R + r]],
                    rows_v.at[b].at[pl.ds(r * IDX_PER_ROW, IDX_PER_ROW)],
                    sem_g[b],
                )

        def drain(sem):
            pltpu.make_async_copy(
                table_hbm.at[pl.ds(0, CHUNK)], rows_v.at[0], sem
            ).wait()

        fire_gathers(0, 0)

        def outer(go, carry):
            for b in range(2):
                g = 2 * go + b
                bn = 1 - b

                @pl.when(g < chunks - 1)
                def _():
                    @pl.when(g >= 1)
                    def _():
                        drain(sem_w[bn])

                    fire_gathers(g + 1, bn)

                drain(sem_g[b])

                def mul_body(i, c2):
                    for j in range(D_MODEL // 16):
                        sl = pl.ds(j * 16, 16)
                        rows_v[b, i, sl] = rows_v[b, i, sl] * SCALE
                    return c2

                lax.fori_loop(0, CHUNK, mul_body, 0, unroll=4)
                pltpu.async_copy(
                    rows_v.at[b],
                    out_hbm.at[pl.ds((row0 + g * R) * IDX_PER_ROW, CHUNK)],
                    sem_w[b],
                )
            return carry

        lax.fori_loop(0, chunks // 2, outer, 0)
        drain(sem_w[0])
        drain(sem_w[1])

    return gather_scale


def kernel(x, lut):
    b, t = x.shape
    flat = x.reshape(-1).astype(jnp.int32)
    n_rows = flat.shape[0] // IDX_PER_ROW
    idx = flat.reshape(n_rows, IDX_PER_ROW)
    out = _make_gather(n_rows)(idx, lut)
    return out.reshape(b, t, D_MODEL)
